# baseline (speedup 1.0000x reference)
import functools

import jax
import jax.numpy as jnp
from jax import lax
from jax.experimental import pallas as pl
from jax.experimental.pallas import tpu as pltpu
from jax.experimental.pallas import tpu_sc as plsc

_NUM_WORKERS = 32
_RUNS_PER_CHUNK = 5
_L = 16


def _ssp(x):
    return jnp.maximum(x, 0.0) + jnp.log1p(jnp.exp(-jnp.abs(x))) - 0.6931471805599453


def _segment_sum_sc(aev2, idx0, idx1, n_feat, n_atoms):
    P = idx0.shape[0]
    EL = 128
    RS = 1024
    n_runs = P // EL
    FPT = 8
    NQ = 4
    runs_pq = n_runs // NQ
    n_chunks = runs_pq // _RUNS_PER_CHUNK
    chunk_edges = _RUNS_PER_CHUNK * EL
    chunk_words = _RUNS_PER_CHUNK * RS
    groups_per_run = EL // _L

    mesh = plsc.VectorSubcoreMesh(core_axis_name="c", subcore_axis_name="s")

    scratch = [
        *[pltpu.VMEM((n_atoms,), jnp.float32) for _ in range(FPT)],
        pltpu.VMEM((chunk_words,), jnp.float32),
        pltpu.VMEM((chunk_words,), jnp.float32),
        pltpu.VMEM((chunk_edges,), jnp.int32),
        pltpu.VMEM((chunk_edges,), jnp.int32),
        pltpu.VMEM((chunk_edges,), jnp.int32),
        pltpu.VMEM((chunk_edges,), jnp.int32),
        pltpu.SemaphoreType.DMA,
        pltpu.SemaphoreType.DMA,
    ]

    @functools.partial(
        pl.kernel,
        mesh=mesh,
        out_type=jax.ShapeDtypeStruct((n_feat, NQ, n_atoms), jnp.float32),
        scratch_types=scratch,
        compiler_params=pltpu.CompilerParams(use_tc_tiling_on_sc=False,
                                             needs_layout_passes=False),
    )
    def seg(aev_hbm, i0_hbm, i1_hbm, out_hbm,
            acc0, acc1, acc2, acc3, acc4, acc5, acc6, acc7,
            vb0, vb1, i0b0, i0b1, i1b0, i1b1, sem0, sem1):
        accs = (acc0, acc1, acc2, acc3, acc4, acc5, acc6, acc7)
        cid = lax.axis_index("c")
        sid = lax.axis_index("s")
        wid = sid * 2 + cid
        fh = wid // NQ
        q = wid - fh * NQ
        run0 = q * runs_pq
        edge0 = run0 * EL

        zeros = jnp.zeros((_L,), jnp.float32)

        def zbody(i, carry):
            for j in range(FPT):
                accs[j][pl.ds(i * _L, _L)] = zeros
            return carry

        lax.fori_loop(0, n_atoms // _L, zbody, 0)

        vbufs = (vb0, vb1)
        i0bufs = (i0b0, i0b1)
        i1bufs = (i1b0, i1b1)
        sems = (sem0, sem1)

        def start_chunk(c, buf):
            sem = sems[buf]
            woff = pl.multiple_of((run0 + c * _RUNS_PER_CHUNK) * RS, 8)
            eoff = pl.multiple_of(edge0 + c * chunk_edges, 8)
            pltpu.make_async_copy(
                aev_hbm.at[fh, pl.ds(woff, chunk_words)],
                vbufs[buf], sem).start()
            pltpu.make_async_copy(
                i0_hbm.at[pl.ds(eoff, chunk_edges)], i0bufs[buf], sem).start()
            pltpu.make_async_copy(
                i1_hbm.at[pl.ds(eoff, chunk_edges)], i1bufs[buf], sem).start()

        def wait_chunk(buf):
            sem = sems[buf]
            pltpu.make_async_copy(aev_hbm.at[0, pl.ds(0, chunk_words)],
                                  vbufs[buf], sem).wait()
            pltpu.make_async_copy(i0_hbm.at[pl.ds(0, chunk_edges)],
                                  i0bufs[buf], sem).wait()
            pltpu.make_async_copy(i0_hbm.at[pl.ds(0, chunk_edges)],
                                  i1bufs[buf], sem).wait()

        def process_chunk(buf):
            vab = vbufs[buf]
            ib0, ib1 = i0bufs[buf], i1bufs[buf]

            def rbody(r, carry):
                vbase = r * RS
                ibase = r * EL
                for k in range(groups_per_run):
                    vi0 = ib0[pl.ds(ibase + k * _L, _L)]
                    vi1 = ib1[pl.ds(ibase + k * _L, _L)]
                    for j in range(FPT):
                        vv = vab[pl.ds(vbase + j * EL + k * _L, _L)]
                        plsc.addupdate_scatter(accs[j], [vi0], vv)
                        plsc.addupdate_scatter(accs[j], [vi1], vv)
                return carry

            lax.fori_loop(0, _RUNS_PER_CHUNK, rbody, 0)

        start_chunk(0, 0)

        def pbody(i, carry):
            c0 = i * 2
            start_chunk(c0 + 1, 1)
            wait_chunk(0)
            process_chunk(0)

            @pl.when(c0 + 2 < n_chunks)
            def _():
                start_chunk(c0 + 2, 0)

            wait_chunk(1)
            process_chunk(1)
            return carry

        lax.fori_loop(0, n_chunks // 2, pbody, 0)
        if n_chunks % 2:
            wait_chunk(0)
            process_chunk(0)

        for j in range(FPT):
            pltpu.sync_copy(accs[j], out_hbm.at[fh * FPT + j, q])

    return seg(aev2, idx0, idx1)


def _dense_body(x_ref, t0_ref, t1_ref, t2_ref, t3_ref,
                Wi_ref, bi_ref, Wj_ref, bj_ref,
                Wint_ref, bint_ref, Wout_ref, bout_ref, Wg_ref, gvec_ref,
                rIW1_ref, rIb1_ref, rIW2_ref, rIb2_ref,
                rAW1_ref, rAb1_ref, rAW2_ref, rAb2_ref,
                rOW1_ref, rOb1_ref, rOW2_ref, rOb2_ref,
                out_e_ref, out_f_ref):
    f32 = jnp.float32
    x = x_ref[...]
    a = _ssp(x)
    proto_no = _ssp(jnp.dot(a, Wi_ref[...], preferred_element_type=f32)
                    + bi_ref[...])
    hj = _ssp(jnp.dot(a, Wj_ref[...], preferred_element_type=f32)
              + bj_ref[...])
    t = (t0_ref[...] + t1_ref[...]) + (t2_ref[...] + t3_ref[...])
    s = jnp.dot(t, Wg_ref[...], preferred_element_type=f32)
    m = hj * s + proto_no

    def res_stack(v, W1_ref, b1_ref, W2_ref, b2_ref):
        for i in range(W1_ref.shape[0]):
            h = jnp.dot(_ssp(v), W1_ref[i], preferred_element_type=f32) \
                + b1_ref[i]
            v = jnp.dot(_ssp(h), W2_ref[i], preferred_element_type=f32) \
                + b2_ref[i] + v
        return v

    m = res_stack(m, rIW1_ref, rIb1_ref, rIW2_ref, rIb2_ref)
    y = x * gvec_ref[...] \
        + jnp.dot(_ssp(m), Wint_ref[...], preferred_element_type=f32) \
        + bint_ref[...]
    y = res_stack(y, rAW1_ref, rAb1_ref, rAW2_ref, rAb2_ref)
    out_f_ref[...] = y
    z = res_stack(y, rOW1_ref, rOb1_ref, rOW2_ref, rOb2_ref)
    e = jnp.dot(_ssp(z), Wout_ref[...], preferred_element_type=f32) \
        + bout_ref[...]
    out_e_ref[...] = e


def _dense_tc(features, tparts, Wi, bi, Wj, bj, Wint, bint, Wout, bout,
              Wg, gvec, rIW1, rIb1, rIW2, rIb2, rAW1, rAb1, rAW2, rAb2,
              rOW1, rOb1, rOW2, rOb2, interpret=False):
    N, F = features.shape
    R = tparts[0].shape[1]
    B = 2000
    grid = (N // B,)

    def rowblk(shape):
        return pl.BlockSpec(shape, lambda i: (i,) + (0,) * (len(shape) - 1))

    def full(arr):
        shape = arr.shape
        return pl.BlockSpec(shape, lambda i, _s=len(shape): (0,) * _s)

    in_specs = [
        rowblk((B, F)),
        rowblk((B, R)), rowblk((B, R)), rowblk((B, R)), rowblk((B, R)),
        full(Wi), full(bi), full(Wj), full(bj), full(Wint), full(bint),
        full(Wout), full(bout), full(Wg), full(gvec),
        full(rIW1), full(rIb1), full(rIW2), full(rIb2),
        full(rAW1), full(rAb1), full(rAW2), full(rAb2),
        full(rOW1), full(rOb1), full(rOW2), full(rOb2),
    ]
    out_specs = [rowblk((B, 1)), rowblk((B, F))]
    out_shape = [
        jax.ShapeDtypeStruct((N, 1), jnp.float32),
        jax.ShapeDtypeStruct((N, F), jnp.float32),
    ]
    return pl.pallas_call(
        _dense_body,
        grid=grid,
        in_specs=in_specs,
        out_specs=out_specs,
        out_shape=out_shape,
        interpret=interpret,
    )(features, *tparts, Wi, bi, Wj, bj, Wint, bint, Wout, bout, Wg, gvec,
      rIW1, rIb1, rIW2, rIb2, rAW1, rAb1, rAW2, rAb2, rOW1, rOb1, rOW2, rOb2)


def kernel(species, features, radial_aev, atom_index12, Wi, bi, Wj, bj,
           Wint, bint, Wout, bout, Wg, gvec, rIW1, rIb1, rIW2, rIb2,
           rAW1, rAb1, rAW2, rAb2, rOW1, rOb1, rOW2, rOb2):
    N, F = features.shape
    P, R = radial_aev.shape

    aev2 = radial_aev.T.reshape(R // 8, 8, P // 128, 128) \
        .transpose(0, 2, 1, 3).reshape(R // 8, (P // 128) * 8 * 128)
    idx0 = atom_index12[0]
    idx1 = atom_index12[1]
    tt4 = _segment_sum_sc(aev2, idx0, idx1, R, N)
    tparts = [tt4[:, i, :].T for i in range(4)]

    out_e, out_f = _dense_tc(
        features, tparts,
        Wi, bi.reshape(1, F), Wj, bj.reshape(1, F),
        Wint, bint.reshape(1, F), Wout, bout.reshape(1, 1),
        Wg, gvec.reshape(1, F),
        rIW1, rIb1.reshape(-1, 1, F), rIW2, rIb2.reshape(-1, 1, F),
        rAW1, rAb1.reshape(-1, 1, F), rAW2, rAb2.reshape(-1, 1, F),
        rOW1, rOb1.reshape(-1, 1, F), rOW2, rOb2.reshape(-1, 1, F))
    return out_e.reshape(species.shape), out_f


# speedup vs baseline: 1.2178x; 1.2178x over previous
import functools

import jax
import jax.numpy as jnp
from jax import lax
from jax.experimental import pallas as pl
from jax.experimental.pallas import tpu as pltpu
from jax.experimental.pallas import tpu_sc as plsc

_NUM_WORKERS = 32
_RUNS_PER_CHUNK = 5
_L = 16


def _ssp(x):
    return jnp.maximum(x, 0.0) + jnp.log1p(jnp.exp(-jnp.abs(x))) - 0.6931471805599453


def _segment_sum_sc(aev2, idx0, idx1, n_feat, n_atoms):
    P = idx0.shape[0]
    EL = 128
    RS = 1024
    n_runs = P // EL
    FPT = 8
    NQ = 4
    runs_pq = n_runs // NQ
    n_chunks = runs_pq // _RUNS_PER_CHUNK
    chunk_edges = _RUNS_PER_CHUNK * EL
    chunk_words = _RUNS_PER_CHUNK * RS
    groups_per_run = EL // _L

    mesh = plsc.VectorSubcoreMesh(core_axis_name="c", subcore_axis_name="s")

    scratch = [
        *[pltpu.VMEM((n_atoms,), jnp.float32) for _ in range(FPT)],
        pltpu.VMEM((chunk_words,), jnp.float32),
        pltpu.VMEM((chunk_words,), jnp.float32),
        pltpu.VMEM((chunk_edges,), jnp.int32),
        pltpu.VMEM((chunk_edges,), jnp.int32),
        pltpu.VMEM((chunk_edges,), jnp.int32),
        pltpu.VMEM((chunk_edges,), jnp.int32),
        pltpu.SemaphoreType.DMA,
        pltpu.SemaphoreType.DMA,
    ]

    @functools.partial(
        pl.kernel,
        mesh=mesh,
        out_type=jax.ShapeDtypeStruct((n_feat, NQ, n_atoms), jnp.float32),
        scratch_types=scratch,
        compiler_params=pltpu.CompilerParams(use_tc_tiling_on_sc=False,
                                             needs_layout_passes=False),
    )
    def seg(aev_hbm, i0_hbm, i1_hbm, out_hbm,
            acc0, acc1, acc2, acc3, acc4, acc5, acc6, acc7,
            vb0, vb1, i0b0, i0b1, i1b0, i1b1, sem0, sem1):
        accs = (acc0, acc1, acc2, acc3, acc4, acc5, acc6, acc7)
        cid = lax.axis_index("c")
        sid = lax.axis_index("s")
        wid = sid * 2 + cid
        fh = wid // NQ
        q = wid - fh * NQ
        run0 = q * runs_pq
        edge0 = run0 * EL

        zeros = jnp.zeros((_L,), jnp.float32)

        @plsc.parallel_loop(0, n_atoms // _L)
        def _zero(i):
            for j in range(FPT):
                accs[j][pl.ds(i * _L, _L)] = zeros

        vbufs = (vb0, vb1)
        i0bufs = (i0b0, i0b1)
        i1bufs = (i1b0, i1b1)
        sems = (sem0, sem1)

        def start_chunk(c, buf):
            sem = sems[buf]
            woff = pl.multiple_of((run0 + c * _RUNS_PER_CHUNK) * RS, 8)
            eoff = pl.multiple_of(edge0 + c * chunk_edges, 8)
            pltpu.make_async_copy(
                aev_hbm.at[fh, pl.ds(woff, chunk_words)],
                vbufs[buf], sem).start()
            pltpu.make_async_copy(
                i0_hbm.at[pl.ds(eoff, chunk_edges)], i0bufs[buf], sem).start()
            pltpu.make_async_copy(
                i1_hbm.at[pl.ds(eoff, chunk_edges)], i1bufs[buf], sem).start()

        def wait_chunk(buf):
            sem = sems[buf]
            pltpu.make_async_copy(aev_hbm.at[0, pl.ds(0, chunk_words)],
                                  vbufs[buf], sem).wait()
            pltpu.make_async_copy(i0_hbm.at[pl.ds(0, chunk_edges)],
                                  i0bufs[buf], sem).wait()
            pltpu.make_async_copy(i0_hbm.at[pl.ds(0, chunk_edges)],
                                  i1bufs[buf], sem).wait()

        def process_chunk(buf):
            vab = vbufs[buf]
            ib0, ib1 = i0bufs[buf], i1bufs[buf]

            @plsc.parallel_loop(0, _RUNS_PER_CHUNK * groups_per_run)
            def _scat(g):
                r = g // groups_per_run
                k = g - r * groups_per_run
                vbase = r * RS + k * _L
                ibase = r * EL + k * _L
                vi0 = ib0[pl.ds(ibase, _L)]
                vi1 = ib1[pl.ds(ibase, _L)]
                for j in range(FPT):
                    vv = vab[pl.ds(vbase + j * EL, _L)]
                    plsc.addupdate_scatter(accs[j], [vi0], vv)
                    plsc.addupdate_scatter(accs[j], [vi1], vv)

        start_chunk(0, 0)

        def pbody(i, carry):
            c0 = i * 2
            start_chunk(c0 + 1, 1)
            wait_chunk(0)
            process_chunk(0)

            @pl.when(c0 + 2 < n_chunks)
            def _():
                start_chunk(c0 + 2, 0)

            wait_chunk(1)
            process_chunk(1)
            return carry

        lax.fori_loop(0, n_chunks // 2, pbody, 0)
        if n_chunks % 2:
            wait_chunk(0)
            process_chunk(0)

        for j in range(FPT):
            pltpu.sync_copy(accs[j], out_hbm.at[fh * FPT + j, q])

    return seg(aev2, idx0, idx1)


def _dense_body(x_ref, t0_ref, t1_ref, t2_ref, t3_ref,
                Wi_ref, bi_ref, Wj_ref, bj_ref,
                Wint_ref, bint_ref, Wout_ref, bout_ref, Wg_ref, gvec_ref,
                rIW1_ref, rIb1_ref, rIW2_ref, rIb2_ref,
                rAW1_ref, rAb1_ref, rAW2_ref, rAb2_ref,
                rOW1_ref, rOb1_ref, rOW2_ref, rOb2_ref,
                out_e_ref, out_f_ref):
    f32 = jnp.float32
    x = x_ref[...]
    a = _ssp(x)
    proto_no = _ssp(jnp.dot(a, Wi_ref[...], preferred_element_type=f32)
                    + bi_ref[...])
    hj = _ssp(jnp.dot(a, Wj_ref[...], preferred_element_type=f32)
              + bj_ref[...])
    t = (t0_ref[...] + t1_ref[...]) + (t2_ref[...] + t3_ref[...])
    s = jnp.dot(t, Wg_ref[...], preferred_element_type=f32)
    m = hj * s + proto_no

    def res_stack(v, W1_ref, b1_ref, W2_ref, b2_ref):
        for i in range(W1_ref.shape[0]):
            h = jnp.dot(_ssp(v), W1_ref[i], preferred_element_type=f32) \
                + b1_ref[i]
            v = jnp.dot(_ssp(h), W2_ref[i], preferred_element_type=f32) \
                + b2_ref[i] + v
        return v

    m = res_stack(m, rIW1_ref, rIb1_ref, rIW2_ref, rIb2_ref)
    y = x * gvec_ref[...] \
        + jnp.dot(_ssp(m), Wint_ref[...], preferred_element_type=f32) \
        + bint_ref[...]
    y = res_stack(y, rAW1_ref, rAb1_ref, rAW2_ref, rAb2_ref)
    out_f_ref[...] = y
    z = res_stack(y, rOW1_ref, rOb1_ref, rOW2_ref, rOb2_ref)
    e = jnp.dot(_ssp(z), Wout_ref[...], preferred_element_type=f32) \
        + bout_ref[...]
    out_e_ref[...] = e


def _dense_tc(features, tparts, Wi, bi, Wj, bj, Wint, bint, Wout, bout,
              Wg, gvec, rIW1, rIb1, rIW2, rIb2, rAW1, rAb1, rAW2, rAb2,
              rOW1, rOb1, rOW2, rOb2, interpret=False):
    N, F = features.shape
    R = tparts[0].shape[1]
    B = 2000
    grid = (N // B,)

    def rowblk(shape):
        return pl.BlockSpec(shape, lambda i: (i,) + (0,) * (len(shape) - 1))

    def full(arr):
        shape = arr.shape
        return pl.BlockSpec(shape, lambda i, _s=len(shape): (0,) * _s)

    in_specs = [
        rowblk((B, F)),
        rowblk((B, R)), rowblk((B, R)), rowblk((B, R)), rowblk((B, R)),
        full(Wi), full(bi), full(Wj), full(bj), full(Wint), full(bint),
        full(Wout), full(bout), full(Wg), full(gvec),
        full(rIW1), full(rIb1), full(rIW2), full(rIb2),
        full(rAW1), full(rAb1), full(rAW2), full(rAb2),
        full(rOW1), full(rOb1), full(rOW2), full(rOb2),
    ]
    out_specs = [rowblk((B, 1)), rowblk((B, F))]
    out_shape = [
        jax.ShapeDtypeStruct((N, 1), jnp.float32),
        jax.ShapeDtypeStruct((N, F), jnp.float32),
    ]
    return pl.pallas_call(
        _dense_body,
        grid=grid,
        in_specs=in_specs,
        out_specs=out_specs,
        out_shape=out_shape,
        interpret=interpret,
    )(features, *tparts, Wi, bi, Wj, bj, Wint, bint, Wout, bout, Wg, gvec,
      rIW1, rIb1, rIW2, rIb2, rAW1, rAb1, rAW2, rAb2, rOW1, rOb1, rOW2, rOb2)


def kernel(species, features, radial_aev, atom_index12, Wi, bi, Wj, bj,
           Wint, bint, Wout, bout, Wg, gvec, rIW1, rIb1, rIW2, rIb2,
           rAW1, rAb1, rAW2, rAb2, rOW1, rOb1, rOW2, rOb2):
    N, F = features.shape
    P, R = radial_aev.shape

    aev2 = radial_aev.T.reshape(R // 8, 8, P // 128, 128) \
        .transpose(0, 2, 1, 3).reshape(R // 8, (P // 128) * 8 * 128)
    idx0 = atom_index12[0]
    idx1 = atom_index12[1]
    tt4 = _segment_sum_sc(aev2, idx0, idx1, R, N)
    tparts = [tt4[:, i, :].T for i in range(4)]

    out_e, out_f = _dense_tc(
        features, tparts,
        Wi, bi.reshape(1, F), Wj, bj.reshape(1, F),
        Wint, bint.reshape(1, F), Wout, bout.reshape(1, 1),
        Wg, gvec.reshape(1, F),
        rIW1, rIb1.reshape(-1, 1, F), rIW2, rIb2.reshape(-1, 1, F),
        rAW1, rAb1.reshape(-1, 1, F), rAW2, rAb2.reshape(-1, 1, F),
        rOW1, rOb1.reshape(-1, 1, F), rOW2, rOb2.reshape(-1, 1, F))
    return out_e.reshape(species.shape), out_f
